# baseline (speedup 1.0000x reference)
import jax
import jax.numpy as jnp
from jax.experimental import pallas as pl
from jax.experimental.pallas import tpu as pltpu


def _ffn_body(x_ref, w1_ref, b1_ref, w2_ref, b2_ref, o_ref):
    xb = x_ref[...].astype(jnp.bfloat16)
    h = jnp.dot(xb, w1_ref[...], preferred_element_type=jnp.float32)
    h = jnp.maximum(h + b1_ref[...], 0.0)
    y = jnp.dot(h.astype(jnp.bfloat16), w2_ref[...],
                preferred_element_type=jnp.float32)
    o_ref[...] = (y + b2_ref[...]).astype(o_ref.dtype)


def kernel(x, w1, b1, w2, b2):
    batch, seq, d_model = x.shape
    d_hidden = w1.shape[1]
    n = batch * seq

    row_tile = 256
    while n % row_tile:
        row_tile //= 2

    x2d = x.reshape(n, d_model)
    w1b = w1.astype(jnp.bfloat16)
    w2b = w2.astype(jnp.bfloat16)
    b1r = b1.reshape(1, d_hidden)
    b2r = b2.reshape(1, d_model)

    flops = 2 * 2 * n * d_model * d_hidden
    bytes_accessed = (4 * n * d_model * 2
                      + 2 * d_model * d_hidden * 2
                      + 4 * (d_hidden + d_model))
    cost = pl.CostEstimate(flops=int(flops), transcendentals=0,
                           bytes_accessed=int(bytes_accessed))

    out2d = pl.pallas_call(
        _ffn_body,
        out_shape=jax.ShapeDtypeStruct((n, d_model), x.dtype),
        grid=(n // row_tile,),
        in_specs=[
            pl.BlockSpec((row_tile, d_model), lambda i: (i, 0)),
            pl.BlockSpec((d_model, d_hidden), lambda i: (0, 0)),
            pl.BlockSpec((1, d_hidden), lambda i: (0, 0)),
            pl.BlockSpec((d_hidden, d_model), lambda i: (0, 0)),
            pl.BlockSpec((1, d_model), lambda i: (0, 0)),
        ],
        out_specs=pl.BlockSpec((row_tile, d_model), lambda i: (i, 0)),
        compiler_params=pltpu.CompilerParams(
            dimension_semantics=("parallel",),
            vmem_limit_bytes=64 * 1024 * 1024,
        ),
        cost_estimate=cost,
    )(x2d, w1b, b1r, w2b, b2r)

    return out2d.reshape(batch, seq, d_model)


# speedup vs baseline: 1.0460x; 1.0460x over previous
import jax
import jax.numpy as jnp
from jax.experimental import pallas as pl
from jax.experimental.pallas import tpu as pltpu


def _ffn_body(x_ref, w1_ref, b1_ref, w2_ref, b2_ref, o_ref):
    xb = x_ref[...].astype(jnp.bfloat16)
    h = jnp.dot(xb, w1_ref[...], preferred_element_type=jnp.float32)
    h = jnp.maximum(h + b1_ref[...], 0.0)
    y = jnp.dot(h.astype(jnp.bfloat16), w2_ref[...],
                preferred_element_type=jnp.float32)
    o_ref[...] = (y + b2_ref[...]).astype(o_ref.dtype)


def kernel(x, w1, b1, w2, b2):
    batch, seq, d_model = x.shape
    d_hidden = w1.shape[1]
    n = batch * seq

    row_tile = 1024
    while n % row_tile:
        row_tile //= 2

    x2d = x.reshape(n, d_model)
    w1b = w1.astype(jnp.bfloat16)
    w2b = w2.astype(jnp.bfloat16)
    b1r = b1.reshape(1, d_hidden)
    b2r = b2.reshape(1, d_model)

    flops = 2 * 2 * n * d_model * d_hidden
    bytes_accessed = (4 * n * d_model * 2
                      + 2 * d_model * d_hidden * 2
                      + 4 * (d_hidden + d_model))
    cost = pl.CostEstimate(flops=int(flops), transcendentals=0,
                           bytes_accessed=int(bytes_accessed))

    out2d = pl.pallas_call(
        _ffn_body,
        out_shape=jax.ShapeDtypeStruct((n, d_model), x.dtype),
        grid=(n // row_tile,),
        in_specs=[
            pl.BlockSpec((row_tile, d_model), lambda i: (i, 0)),
            pl.BlockSpec((d_model, d_hidden), lambda i: (0, 0)),
            pl.BlockSpec((1, d_hidden), lambda i: (0, 0)),
            pl.BlockSpec((d_hidden, d_model), lambda i: (0, 0)),
            pl.BlockSpec((1, d_model), lambda i: (0, 0)),
        ],
        out_specs=pl.BlockSpec((row_tile, d_model), lambda i: (i, 0)),
        compiler_params=pltpu.CompilerParams(
            dimension_semantics=("parallel",),
            vmem_limit_bytes=64 * 1024 * 1024,
        ),
        cost_estimate=cost,
    )(x2d, w1b, b1r, w2b, b2r)

    return out2d.reshape(batch, seq, d_model)
